# baseline (speedup 1.0000x reference)
import functools

import jax
import jax.numpy as jnp
from jax import lax
from jax.experimental import pallas as pl
from jax.experimental.pallas import tpu as pltpu
from jax.experimental.pallas import tpu_sc as plsc

DIM = 16
NW = 32
BPW = 512
NBB = BPW // 128


def _gather_call(B, L):
    n_flat = B * L * DIM
    l_stride = B * DIM
    mesh = plsc.VectorSubcoreMesh(core_axis_name="c", subcore_axis_name="s")

    @functools.partial(
        pl.kernel,
        mesh=mesh,
        out_type=jax.ShapeDtypeStruct((n_flat,), jnp.float32),
        scratch_types=[
            pltpu.VMEM((2, BPW), jnp.int32),
            pltpu.VMEM((2, BPW, DIM), jnp.float32),
            pltpu.VMEM((2, 2 * NBB * 1024), jnp.float32),
            pltpu.SemaphoreType.DMA,
            pltpu.SemaphoreType.DMA,
            pltpu.SemaphoreType.DMA,
            pltpu.SemaphoreType.DMA,
        ],
        compiler_params=pltpu.CompilerParams(
            use_tc_tiling_on_sc=False, needs_layout_passes=False),
    )
    def emb(idx_hbm, table_hbm, out_hbm, idxbuf, rowbuf, tbuf,
            gsem0, gsem1, osem0, osem1):
        wid = lax.axis_index("s") * 2 + lax.axis_index("c")
        b0 = wid * BPW
        gsems = (gsem0, gsem1)
        osems = (osem0, osem1)
        iota = lax.iota(jnp.int32, 16)
        dpos = (iota // 8) * (NBB * 1024) + (iota % 8) * 128

        def load_and_fire(l, r):
            pltpu.sync_copy(idx_hbm.at[l, pl.ds(b0, BPW)], idxbuf.at[r])

            def add_body(i, c):
                for s in range(8):
                    sl = pl.ds(i * 128 + s * 16, 16)
                    idxbuf[r, sl] = idxbuf[r, sl] + 1
                return c

            lax.fori_loop(0, BPW // 128, add_body, 0)
            for j in range(NBB):
                pltpu.async_copy(
                    table_hbm.at[idxbuf.at[r].at[pl.ds(j * 128, 128)]],
                    rowbuf.at[r].at[pl.ds(j * 128, 128)],
                    gsems[r],
                )

        def drain_gathers(r):
            pltpu.make_async_copy(
                table_hbm.at[pl.ds(0, BPW)], rowbuf.at[r], gsems[r]
            ).wait()

        def transpose_and_store(l, r):
            def tr_body(k, c):
                bb = k // 8
                bg = k % 8
                base = dpos + (bb * 1024 + bg * 16)
                roff = bb * 128 + bg * 16
                for j in range(16):
                    row = rowbuf[r, roff + j, :]
                    plsc.store_scatter(tbuf.at[r], [base + j], row)
                return c

            lax.fori_loop(0, NBB * 8, tr_body, 0)
            off = l * l_stride + wid * (NBB * 1024)
            pltpu.async_copy(
                tbuf.at[r].at[pl.ds(0, NBB * 1024)],
                out_hbm.at[pl.ds(off, NBB * 1024)], osems[r])
            pltpu.async_copy(
                tbuf.at[r].at[pl.ds(NBB * 1024, NBB * 1024)],
                out_hbm.at[pl.ds(off + (B * DIM // 2), NBB * 1024)], osems[r])

        def wait_store(l, r):
            off = l * l_stride + wid * (NBB * 1024)
            pltpu.make_async_copy(
                tbuf.at[r], out_hbm.at[pl.ds(off, 2 * NBB * 1024)], osems[r]
            ).wait()

        load_and_fire(0, 0)

        def half_body(l, r):
            @pl.when(l < L - 1)
            def _():
                load_and_fire(l + 1, 1 - r)

            drain_gathers(r)

            @pl.when(l >= 2)
            def _():
                wait_store(l - 2, r)

            transpose_and_store(l, r)

        def pair_body(g, carry):
            half_body(2 * g, 0)
            half_body(2 * g + 1, 1)
            return carry

        lax.fori_loop(0, L // 2, pair_body, 0)
        wait_store(L - 2, 0)
        wait_store(L - 1, 1)

    return emb


def _retag_call(B, L):
    n_tiles = B * L * DIM // 1024
    tiles_per_w = n_tiles // NW
    tiles_per_l = B * DIM // 1024
    mesh = plsc.VectorSubcoreMesh(core_axis_name="c", subcore_axis_name="s")

    @functools.partial(
        pl.kernel,
        mesh=mesh,
        out_type=jax.ShapeDtypeStruct((L, DIM, B), jnp.float32),
        scratch_types=[
            pltpu.VMEM((2, 8, 128), jnp.float32),
            pltpu.SemaphoreType.DMA,
            pltpu.SemaphoreType.DMA,
            pltpu.SemaphoreType.DMA,
        ],
        compiler_params=pltpu.CompilerParams(use_tc_tiling_on_sc=True),
    )
    def retag(in_hbm, out_hbm, buf, lsem, sem0, sem1):
        wid = lax.axis_index("s") * 2 + lax.axis_index("c")
        t0 = wid * tiles_per_w
        sems = (sem0, sem1)

        def out_slice(t):
            l = t // tiles_per_l
            rem = t % tiles_per_l
            dh = rem // (tiles_per_l // 2)
            bb = rem % (tiles_per_l // 2)
            return out_hbm.at[l, pl.ds(dh * 8, 8), pl.ds(bb * 128, 128)]

        def copy_tile(t, r):
            @pl.when(t >= t0 + 2)
            def _():
                pltpu.make_async_copy(buf.at[r], out_slice(t - 2), sems[r]).wait()

            pltpu.async_copy(in_hbm.at[pl.ds(t * 8, 8)], buf.at[r], lsem).wait()
            pltpu.async_copy(buf.at[r], out_slice(t), sems[r])

        def pair_body(g, carry):
            copy_tile(t0 + 2 * g, 0)
            copy_tile(t0 + 2 * g + 1, 1)
            return carry

        lax.fori_loop(0, tiles_per_w // 2, pair_body, 0)
        for r in range(2):
            t = t0 + tiles_per_w - 2 + r
            pltpu.make_async_copy(buf.at[r], out_slice(t), sems[r]).wait()

    return retag


def kernel(action, table):
    B, L, _ = action.shape
    idx2d = jnp.swapaxes(action.squeeze(-1), 0, 1)
    flat = _gather_call(B, L)(idx2d, table)
    tiled = _retag_call(B, L)(flat.reshape(B * L * DIM // 128, 128))
    return lax.transpose(tiled, (2, 0, 1))
